# baseline (speedup 1.0000x reference)
import functools

import jax
import jax.numpy as jnp
from jax import lax
from jax.experimental import pallas as pl
from jax.experimental.pallas import tpu as pltpu
from jax.experimental.pallas import tpu_sc as plsc

BATCH = 16384
EMBED_DIM = 128
NUM_CORES = 2
NUM_SUBCORES = 16
NUM_WORKERS = NUM_CORES * NUM_SUBCORES
B_PER_W = BATCH // NUM_WORKERS
CHUNK = 128
NUM_CHUNKS = B_PER_W // CHUNK
GROUPS_PER_CHUNK = CHUNK // 16


def _body(user_ref, item_ref, eu_ref, ei_ref, out_ref,
          idx_u, idx_i, ubig, ibig, outv, tbuf,
          sems_u, sems_i, sem_iu, sem_ii):
    wid = lax.axis_index("c") * NUM_SUBCORES + lax.axis_index("s")
    base = pl.multiple_of(wid * B_PER_W, B_PER_W)

    cu = pltpu.async_copy(user_ref.at[pl.ds(base, B_PER_W)], idx_u, sem_iu)
    ci = pltpu.async_copy(item_ref.at[pl.ds(base, B_PER_W)], idx_i, sem_ii)
    cu.wait()
    ci.wait()

    iota = lax.iota(jnp.int32, 16)

    def copies(h, p):
        off = pl.multiple_of(h * CHUNK, CHUNK)
        slot = pl.multiple_of(p * CHUNK, CHUNK)
        cu = pltpu.make_async_copy(
            eu_ref.at[idx_u.at[pl.ds(off, CHUNK)]],
            ubig.at[pl.ds(slot, CHUNK)], sems_u.at[p])
        ci = pltpu.make_async_copy(
            ei_ref.at[idx_i.at[pl.ds(off, CHUNK)]],
            ibig.at[pl.ds(slot, CHUNK)], sems_i.at[p])
        return cu, ci

    def fire(h, p):
        cu, ci = copies(h, p)
        cu.start()
        ci.start()

    fire(0, 0)
    fire(1, 1)

    def chunk_body(g, _):
        p = lax.rem(g, 3)

        @pl.when(g + 2 < NUM_CHUNKS)
        def _fire_next():
            fire(g + 2, lax.rem(g + 2, 3))

        cu, ci = copies(g, p)
        cu.wait()
        ci.wait()
        rbase = p * CHUNK

        def group(t, _):
            b0 = t * 16
            for j in range(16):
                row = rbase + b0 + j
                acc = ubig[row, pl.ds(0, 16)] * ibig[row, pl.ds(0, 16)]
                for k in range(1, 8):
                    acc = acc + (ubig[row, pl.ds(16 * k, 16)]
                                 * ibig[row, pl.ds(16 * k, 16)])
                tbuf[pl.ds(17 * j, 16)] = acc
            row17 = iota * 17
            tot = plsc.load_gather(tbuf, [row17])
            for col in range(1, 16):
                tot = tot + plsc.load_gather(tbuf, [row17 + col])
            outv[pl.ds(g * CHUNK + b0, 16)] = tot
            return 0

        lax.fori_loop(0, GROUPS_PER_CHUNK, group, 0)
        return 0

    lax.fori_loop(0, NUM_CHUNKS, chunk_body, 0)

    pltpu.sync_copy(outv, out_ref.at[pl.ds(base, B_PER_W)])


@jax.jit
def _mf(user, item, embed_user, embed_item):
    mesh = plsc.VectorSubcoreMesh(
        core_axis_name="c", subcore_axis_name="s",
        num_cores=NUM_CORES, num_subcores=NUM_SUBCORES)
    return pl.kernel(
        _body,
        out_type=jax.ShapeDtypeStruct((BATCH,), jnp.float32),
        mesh=mesh,
        compiler_params=pltpu.CompilerParams(
            needs_layout_passes=False,
            disable_bounds_checks=True,
            disable_semaphore_checks=True,
        ),
        scratch_types=[
            pltpu.VMEM((B_PER_W,), jnp.int32),
            pltpu.VMEM((B_PER_W,), jnp.int32),
            pltpu.VMEM((3 * CHUNK, EMBED_DIM), jnp.float32),
            pltpu.VMEM((3 * CHUNK, EMBED_DIM), jnp.float32),
            pltpu.VMEM((B_PER_W,), jnp.float32),
            pltpu.VMEM((272,), jnp.float32),
            pltpu.SemaphoreType.DMA((3,)),
            pltpu.SemaphoreType.DMA((3,)),
            pltpu.SemaphoreType.DMA,
            pltpu.SemaphoreType.DMA,
        ],
    )(user, item, embed_user, embed_item)


def kernel(user, item, embed_user, embed_item):
    return _mf(user.astype(jnp.int32), item.astype(jnp.int32),
               embed_user, embed_item)


# speedup vs baseline: 1.0235x; 1.0235x over previous
import functools

import jax
import jax.numpy as jnp
from jax import lax
from jax.experimental import pallas as pl
from jax.experimental.pallas import tpu as pltpu
from jax.experimental.pallas import tpu_sc as plsc

BATCH = 16384
EMBED_DIM = 128
NUM_CORES = 2
NUM_SUBCORES = 16
NUM_WORKERS = NUM_CORES * NUM_SUBCORES
B_PER_W = BATCH // NUM_WORKERS
CHUNK = 128
NUM_CHUNKS = B_PER_W // CHUNK
GROUPS_PER_CHUNK = CHUNK // 16


def _body(user_ref, item_ref, eu_ref, ei_ref, out_ref,
          idx_u, idx_i, ubig, ibig, outv, tbuf,
          sems_u, sems_i, sem_iu, sem_ii):
    wid = lax.axis_index("c") * NUM_SUBCORES + lax.axis_index("s")
    base = pl.multiple_of(wid * B_PER_W, B_PER_W)

    cu = pltpu.async_copy(user_ref.at[pl.ds(base, B_PER_W)], idx_u, sem_iu)
    ci = pltpu.async_copy(item_ref.at[pl.ds(base, B_PER_W)], idx_i, sem_ii)
    cu.wait()
    ci.wait()

    iota = lax.iota(jnp.int32, 16)

    def copies(h, p):
        off = pl.multiple_of(h * CHUNK, CHUNK)
        slot = pl.multiple_of(p * CHUNK, CHUNK)
        cu = pltpu.make_async_copy(
            eu_ref.at[idx_u.at[pl.ds(off, CHUNK)]],
            ubig.at[pl.ds(slot, CHUNK)], sems_u.at[p])
        ci = pltpu.make_async_copy(
            ei_ref.at[idx_i.at[pl.ds(off, CHUNK)]],
            ibig.at[pl.ds(slot, CHUNK)], sems_i.at[p])
        return cu, ci

    def fire(h, p):
        cu, ci = copies(h, p)
        cu.start()
        ci.start()

    fire(0, 0)

    def chunk_body(g, _):
        p = g & 1

        @pl.when(g + 1 < NUM_CHUNKS)
        def _fire_next():
            fire(g + 1, (g + 1) & 1)

        cu, ci = copies(g, p)
        cu.wait()
        ci.wait()
        rbase = p * CHUNK

        def group(t, _):
            b0 = t * 16
            for j in range(16):
                row = rbase + b0 + j
                acc = ubig[row, pl.ds(0, 16)] * ibig[row, pl.ds(0, 16)]
                for k in range(1, 8):
                    acc = acc + (ubig[row, pl.ds(16 * k, 16)]
                                 * ibig[row, pl.ds(16 * k, 16)])
                tbuf[pl.ds(17 * j, 16)] = acc
            row17 = iota * 17
            tot = plsc.load_gather(tbuf, [row17])
            for col in range(1, 16):
                tot = tot + plsc.load_gather(tbuf, [row17 + col])
            outv[pl.ds(g * CHUNK + b0, 16)] = tot
            return 0

        lax.fori_loop(0, GROUPS_PER_CHUNK, group, 0)
        return 0

    lax.fori_loop(0, NUM_CHUNKS, chunk_body, 0)

    pltpu.sync_copy(outv, out_ref.at[pl.ds(base, B_PER_W)])


@jax.jit
def _mf(user, item, embed_user, embed_item):
    mesh = plsc.VectorSubcoreMesh(
        core_axis_name="c", subcore_axis_name="s",
        num_cores=NUM_CORES, num_subcores=NUM_SUBCORES)
    return pl.kernel(
        _body,
        out_type=jax.ShapeDtypeStruct((BATCH,), jnp.float32),
        mesh=mesh,
        compiler_params=pltpu.CompilerParams(
            needs_layout_passes=False,
            disable_bounds_checks=True,
            disable_semaphore_checks=True,
        ),
        scratch_types=[
            pltpu.VMEM((B_PER_W,), jnp.int32),
            pltpu.VMEM((B_PER_W,), jnp.int32),
            pltpu.VMEM((2 * CHUNK, EMBED_DIM), jnp.float32),
            pltpu.VMEM((2 * CHUNK, EMBED_DIM), jnp.float32),
            pltpu.VMEM((B_PER_W,), jnp.float32),
            pltpu.VMEM((272,), jnp.float32),
            pltpu.SemaphoreType.DMA((2,)),
            pltpu.SemaphoreType.DMA((2,)),
            pltpu.SemaphoreType.DMA,
            pltpu.SemaphoreType.DMA,
        ],
    )(user, item, embed_user, embed_item)


def kernel(user, item, embed_user, embed_item):
    return _mf(user.astype(jnp.int32), item.astype(jnp.int32),
               embed_user, embed_item)
